# baseline (speedup 1.0000x reference)
import jax
import jax.numpy as jnp
from jax import lax
from jax.experimental import pallas as pl
from jax.experimental.pallas import tpu as pltpu, tpu_sc as plsc

SCALING = 10.0
DT = 1.0
TIME_STEPS = 51

NC, NS, L = 2, 16, 16
NW = NC * NS
B = 4096
F = 26
C = 4
M = F * C
ROWS_PER_W = B // NW
NCB = M // 8
BROWS = TIME_STEPS + 2


def _sc_body(trace_hbm, center_hbm, out_hbm, trace_v, center_v,
             buf0, buf1, bsave0, bsave1, sem0, sem1):
    wid = lax.axis_index("s") * NC + lax.axis_index("c")
    row0 = wid * ROWS_PER_W
    bufs = (buf0, buf1)
    bsaves = (bsave0, bsave1)
    sems = (sem0, sem1)

    pltpu.sync_copy(center_hbm, center_v)
    pltpu.sync_copy(trace_hbm.at[pl.ds(row0, ROWS_PER_W), :], trace_v)

    def zero_buf(i, _):
        b = i >> 3
        cl = i & 7
        zeros = jnp.zeros((L,), jnp.int32)
        for off in range(0, ROWS_PER_W, L):
            buf0[b, cl, pl.ds(off, L)] = zeros
            buf1[b, cl, pl.ds(off, L)] = zeros
        return 0

    lax.fori_loop(0, BROWS * 8, zero_buf, 0)

    def scatter_ones(cb, slot):
        buf, bsave = bufs[slot], bsaves[slot]

        def one_vec(i, _):
            for u in range(4):
                v = i * 4 + u
                cl = v >> 3
                r = ((v & 7) << 4) + lax.iota(jnp.int32, L)
                f = (cb << 1) + (cl >> 2)
                t = plsc.load_gather(trace_v, [r, r * 0 + f])
                c = plsc.load_gather(center_v, [r * 0 + (cl & 3)])
                times = SCALING * jnp.abs(t - c)
                bins = (times / DT + 1.0).astype(jnp.int32)
                bins = jnp.clip(bins, 0, BROWS - 1)
                bsave[pl.ds(v * L, L)] = bins
                plsc.store_scatter(buf, [bins, r * 0 + cl, r], r * 0 + 1)
            return 0

        lax.fori_loop(0, 16, one_vec, 0)

    def scatter_zeros(slot):
        buf, bsave = bufs[slot], bsaves[slot]

        def one_vec(i, _):
            for u in range(4):
                v = i * 4 + u
                cl = v >> 3
                r = ((v & 7) << 4) + lax.iota(jnp.int32, L)
                bins = bsave[pl.ds(v * L, L)]
                plsc.store_scatter(buf, [bins, r * 0 + cl, r], r * 0)
            return 0

        lax.fori_loop(0, 16, one_vec, 0)

    def dma(cb, slot):
        dst = out_hbm.at[:, pl.ds(cb * 8, 8), pl.ds(row0, ROWS_PER_W)]
        return pltpu.make_async_copy(
            bufs[slot].at[pl.ds(0, TIME_STEPS)], dst, sems[slot])

    for cb in range(NCB):
        slot = cb & 1
        if cb >= 2:
            dma(cb - 2, slot).wait()
            scatter_zeros(slot)
        scatter_ones(cb, slot)
        dma(cb, slot).start()
    dma(NCB - 2, (NCB - 2) & 1).wait()
    dma(NCB - 1, (NCB - 1) & 1).wait()


def kernel(trace, dummy1, dummy2, center):
    del dummy1, dummy2
    mesh = plsc.VectorSubcoreMesh(core_axis_name="c", subcore_axis_name="s")
    run = pl.kernel(
        _sc_body,
        out_type=jax.ShapeDtypeStruct((TIME_STEPS, M, B), jnp.int32),
        mesh=mesh,
        compiler_params=pltpu.CompilerParams(needs_layout_passes=False),
        scratch_types=[
            pltpu.VMEM((ROWS_PER_W, F), jnp.float32),
            pltpu.VMEM((C,), jnp.float32),
            pltpu.VMEM((BROWS, 8, ROWS_PER_W), jnp.int32),
            pltpu.VMEM((BROWS, 8, ROWS_PER_W), jnp.int32),
            pltpu.VMEM((8 * ROWS_PER_W,), jnp.int32),
            pltpu.VMEM((8 * ROWS_PER_W,), jnp.int32),
            pltpu.SemaphoreType.DMA,
            pltpu.SemaphoreType.DMA,
        ],
    )
    return run(trace, center).transpose(0, 2, 1)


# speedup vs baseline: 1.0741x; 1.0741x over previous
import jax
import jax.numpy as jnp
from jax import lax
from jax.experimental import pallas as pl
from jax.experimental.pallas import tpu as pltpu, tpu_sc as plsc

SCALING = 10.0
DT = 1.0
TIME_STEPS = 51

NC, NS, L = 2, 16, 16
NW = NC * NS
B = 4096
F = 26
C = 4
M = F * C
ROWS_PER_W = B // NW
NCB = M // 8
BROWS = TIME_STEPS + 2


def _sc_body(trace_hbm, center_hbm, out_hbm, trace_v, center_v,
             buf0, buf1, bsave0, bsave1, sem0, sem1):
    wid = lax.axis_index("s") * NC + lax.axis_index("c")
    row0 = wid * ROWS_PER_W
    bufs = (buf0, buf1)
    bsaves = (bsave0, bsave1)
    sems = (sem0, sem1)

    pltpu.sync_copy(center_hbm, center_v)
    pltpu.sync_copy(trace_hbm.at[pl.ds(row0 * F, ROWS_PER_W * F)], trace_v)

    def zero_buf(buf):
        def body(i, _):
            b = i >> 3
            cl = i & 7
            zeros = jnp.zeros((L,), jnp.int32)
            for off in range(0, ROWS_PER_W, L):
                buf[b, cl, pl.ds(off, L)] = zeros
            return 0

        lax.fori_loop(0, BROWS * 8, body, 0)

    def pass_cb(cb, slot, rezero):
        buf, bsave = bufs[slot], bsaves[slot]

        def one_vec(i, _):
            for u in range(4):
                v = i * 4 + u
                cl = v >> 3
                r = ((v & 7) << 4) + lax.iota(jnp.int32, L)
                f = (cb << 1) + (cl >> 2)
                if rezero:
                    old = bsave[pl.ds(v * L, L)]
                    plsc.store_scatter(buf, [old, r * 0 + cl, r], r * 0)
                t = plsc.load_gather(trace_v, [r * F + f])
                c = plsc.load_gather(center_v, [r * 0 + (cl & 3)])
                times = SCALING * jnp.abs(t - c)
                bins = (times / DT + 1.0).astype(jnp.int32)
                bins = jnp.clip(bins, 0, BROWS - 1)
                bsave[pl.ds(v * L, L)] = bins
                plsc.store_scatter(buf, [bins, r * 0 + cl, r], r * 0 + 1)
            return 0

        lax.fori_loop(0, 16, one_vec, 0)

    def dma(cb, slot):
        dst = out_hbm.at[:, pl.ds(cb * 8, 8), pl.ds(row0, ROWS_PER_W)]
        return pltpu.make_async_copy(
            bufs[slot].at[pl.ds(0, TIME_STEPS)], dst, sems[slot])

    zero_buf(buf0)
    pass_cb(0, 0, rezero=False)
    dma(0, 0).start()
    zero_buf(buf1)
    pass_cb(1, 1, rezero=False)
    dma(1, 1).start()
    for cb in range(2, NCB):
        slot = cb & 1
        dma(cb - 2, slot).wait()
        pass_cb(cb, slot, rezero=True)
        dma(cb, slot).start()
    dma(NCB - 2, (NCB - 2) & 1).wait()
    dma(NCB - 1, (NCB - 1) & 1).wait()


def kernel(trace, dummy1, dummy2, center):
    del dummy1, dummy2
    trace_flat = trace.reshape(-1)
    mesh = plsc.VectorSubcoreMesh(core_axis_name="c", subcore_axis_name="s")
    run = pl.kernel(
        _sc_body,
        out_type=jax.ShapeDtypeStruct((TIME_STEPS, M, B), jnp.int32),
        mesh=mesh,
        compiler_params=pltpu.CompilerParams(needs_layout_passes=False),
        scratch_types=[
            pltpu.VMEM((ROWS_PER_W * F,), jnp.float32),
            pltpu.VMEM((C,), jnp.float32),
            pltpu.VMEM((BROWS, 8, ROWS_PER_W), jnp.int32),
            pltpu.VMEM((BROWS, 8, ROWS_PER_W), jnp.int32),
            pltpu.VMEM((8 * ROWS_PER_W,), jnp.int32),
            pltpu.VMEM((8 * ROWS_PER_W,), jnp.int32),
            pltpu.SemaphoreType.DMA,
            pltpu.SemaphoreType.DMA,
        ],
    )
    return run(trace_flat, center).transpose(0, 2, 1)
